# baseline (speedup 1.0000x reference)
import jax
import jax.numpy as jnp
from jax.experimental import pallas as pl
from jax.experimental.pallas import tpu as pltpu

_F32 = jnp.float32


def _make_body(B, N, P, D, DOUT):
    PB = P * N
    M = B * PB
    C = 3 * DOUT

    def body(xs_ref, xo_ref, wx_ref, wn_ref, wr_ref,
             bx_ref, bn_ref, br_ref, g_ref, bt_ref,
             o_ref, hs_ref, sacc_ref, qacc_ref):
        j = pl.program_id(1)

        @pl.when(j == 0)
        def _compute():
            atts = []
            for b in range(B):
                num = jnp.zeros((N, N), _F32)
                sq1 = jnp.zeros((N, 1), _F32)
                sq2 = jnp.zeros((N, 1), _F32)
                for p in range(P):
                    r = b * PB + p * N
                    a1 = xs_ref[0, r:r + N, :]
                    a2 = xo_ref[0, r:r + N, :]
                    num += jax.lax.dot_general(
                        a1, a2, (((1,), (1,)), ((), ())),
                        preferred_element_type=_F32)
                    sq1 += jnp.sum(a1 * a1, axis=1, keepdims=True)
                    sq2 += jnp.sum(a2 * a2, axis=1, keepdims=True)
                n1 = jnp.maximum(jnp.sqrt(sq1), 1e-6)
                n2 = jnp.maximum(jnp.sqrt(sq2), 1e-6)
                sim = num / (n1 * n2.T)
                m1 = jnp.max(sim, axis=1, keepdims=True)
                e1 = jnp.exp(sim - m1)
                atts.append(e1 / jnp.sum(e1, axis=1, keepdims=True))

            X = xs_ref[0]
            S = jnp.dot(X, wx_ref[:], preferred_element_type=_F32) + bx_ref[:]
            rn2 = jnp.sum(S * S, axis=1, keepdims=True)

            Z = jnp.dot(X, wn_ref[:], preferred_element_type=_F32)
            Z3 = Z.reshape(B * P, N, DOUT)
            sz = jnp.sum(Z3, axis=1, keepdims=True)
            XN = ((sz - Z3) * (1.0 / (N - 1))).reshape(M, DOUT) + bn_ref[:]
            rn2 += jnp.sum(XN * XN, axis=1, keepdims=True)

            Zs = jnp.dot(X, wr_ref[:], preferred_element_type=_F32)
            Zo = jnp.dot(xo_ref[0], wr_ref[:], preferred_element_type=_F32)
            mu_parts = []
            for b in range(B):
                for p in range(P):
                    r = b * PB + p * N
                    c = jnp.dot(atts[b], Zo[r:r + N, :],
                                preferred_element_type=_F32)
                    mu_parts.append(Zs[r:r + N, :] - c + br_ref[:])
            MUS = jnp.concatenate(mu_parts, axis=0)
            rn2 += jnp.sum(MUS * MUS, axis=1, keepdims=True)

            rinv = 1.0 / jnp.maximum(jnp.sqrt(rn2), 1e-12)
            h = jnp.concatenate([S, XN, MUS], axis=1) * rinv
            h = jnp.maximum(h, 0.0)
            hs_ref[:] = h
            sacc_ref[:] = jnp.sum(h, axis=0, keepdims=True)
            qacc_ref[:] = jnp.sum(h * h, axis=0, keepdims=True)

        @pl.when(j > 0)
        def _writeback():
            mean = sacc_ref[:] * (1.0 / M)
            var = jnp.maximum(qacc_ref[:] * (1.0 / M) - mean * mean, 0.0)
            scale = g_ref[:] * jax.lax.rsqrt(var + 1e-5)
            shift = bt_ref[:] - mean * scale
            h = hs_ref[pl.ds((j - 1) * PB, PB), :] * scale + shift
            for p in range(P):
                o_ref[0, 0, :, p, :] = h[p * N:(p + 1) * N, :]

    return body


def kernel(x1, x2, adj1, adj2, Wx_w, Wx_b, Wn_w, Wn_b, Wr_w, Wr_b, gamma, beta):
    B, N, P, D = x1.shape
    DOUT = Wx_w.shape[0]
    C = 3 * DOUT
    PB = P * N
    M = B * PB

    x1t = x1.transpose(0, 2, 1, 3).reshape(M, D)
    x2t = x2.transpose(0, 2, 1, 3).reshape(M, D)
    XA = jnp.stack([x1t, x2t])

    vspec = pl.BlockSpec(memory_space=pltpu.MemorySpace.VMEM)
    self_spec = pl.BlockSpec((1, M, D), lambda i, j: (i, 0, 0))
    other_spec = pl.BlockSpec((1, M, D), lambda i, j: (1 - i, 0, 0))
    out_spec = pl.BlockSpec(
        (1, 1, N, P, C), lambda i, j: (i, jnp.maximum(j - 1, 0), 0, 0, 0))

    out = pl.pallas_call(
        _make_body(B, N, P, D, DOUT),
        grid=(2, B + 1),
        in_specs=[self_spec, other_spec] + [vspec] * 8,
        out_specs=out_spec,
        out_shape=jax.ShapeDtypeStruct((2, B, N, P, C), jnp.float32),
        scratch_shapes=[
            pltpu.VMEM((M, C), jnp.float32),
            pltpu.VMEM((1, C), jnp.float32),
            pltpu.VMEM((1, C), jnp.float32),
        ],
    )(XA, XA,
      Wx_w.T, Wn_w.T, Wr_w.T,
      Wx_b.reshape(1, DOUT), Wn_b.reshape(1, DOUT), Wr_b.reshape(1, DOUT),
      gamma.reshape(1, C), beta.reshape(1, C))

    return (out[0], out[1])


# speedup vs baseline: 1.4017x; 1.4017x over previous
import jax
import jax.numpy as jnp
from jax.experimental import pallas as pl
from jax.experimental.pallas import tpu as pltpu

_F32 = jnp.float32


def _make_body(B, N, P, D, DOUT):
    PB = P * N
    M = B * PB
    C = 3 * DOUT

    def body(xs_ref, xo_ref, wx_ref, wn_ref, wr_ref,
             bx_ref, bn_ref, br_ref, g_ref, bt_ref,
             o_ref, hs_ref, sacc_ref, qacc_ref):
        j = pl.program_id(1)

        @pl.when(j == 0)
        def _compute():
            atts = []
            for b in range(B):
                num = jnp.zeros((N, N), _F32)
                sq1 = jnp.zeros((N, 1), _F32)
                sq2 = jnp.zeros((N, 1), _F32)
                for p in range(P):
                    r = b * PB + p * N
                    a1 = xs_ref[0, r:r + N, :]
                    a2 = xo_ref[0, r:r + N, :]
                    num += jax.lax.dot_general(
                        a1, a2, (((1,), (1,)), ((), ())),
                        preferred_element_type=_F32)
                    sq1 += jnp.sum(a1 * a1, axis=1, keepdims=True)
                    sq2 += jnp.sum(a2 * a2, axis=1, keepdims=True)
                n1 = jnp.maximum(jnp.sqrt(sq1), 1e-6)
                n2 = jnp.maximum(jnp.sqrt(sq2), 1e-6)
                sim = num / (n1 * n2.T)
                m1 = jnp.max(sim, axis=1, keepdims=True)
                e1 = jnp.exp(sim - m1)
                atts.append(e1 / jnp.sum(e1, axis=1, keepdims=True))

            X = xs_ref[0]
            S = jnp.dot(X, wx_ref[:], preferred_element_type=_F32) + bx_ref[:]
            rn2 = jnp.sum(S * S, axis=1, keepdims=True)

            Z = jnp.dot(X, wn_ref[:], preferred_element_type=_F32)
            Z3 = Z.reshape(B * P, N, DOUT)
            sz = jnp.sum(Z3, axis=1, keepdims=True)
            XN = ((sz - Z3) * (1.0 / (N - 1))).reshape(M, DOUT) + bn_ref[:]
            rn2 += jnp.sum(XN * XN, axis=1, keepdims=True)

            Zs = jnp.dot(X, wr_ref[:], preferred_element_type=_F32)
            Zo = jnp.dot(xo_ref[0], wr_ref[:], preferred_element_type=_F32)
            mu_parts = []
            for b in range(B):
                for p in range(P):
                    r = b * PB + p * N
                    c = jnp.dot(atts[b], Zo[r:r + N, :],
                                preferred_element_type=_F32)
                    mu_parts.append(Zs[r:r + N, :] - c + br_ref[:])
            MUS = jnp.concatenate(mu_parts, axis=0)
            rn2 += jnp.sum(MUS * MUS, axis=1, keepdims=True)

            rinv = 1.0 / jnp.maximum(jnp.sqrt(rn2), 1e-12)
            h = jnp.concatenate([S, XN, MUS], axis=1) * rinv
            h = jnp.maximum(h, 0.0)
            hs_ref[:] = h
            sacc_ref[:] = jnp.sum(h, axis=0, keepdims=True)
            qacc_ref[:] = jnp.sum(h * h, axis=0, keepdims=True)

        @pl.when(j > 0)
        def _writeback():
            mean = sacc_ref[:] * (1.0 / M)
            var = jnp.maximum(qacc_ref[:] * (1.0 / M) - mean * mean, 0.0)
            scale = g_ref[:] * jax.lax.rsqrt(var + 1e-5)
            shift = bt_ref[:] - mean * scale
            h = hs_ref[pl.ds((j - 1) * PB, PB), :] * scale + shift
            o_ref[0] = h

    return body


def kernel(x1, x2, adj1, adj2, Wx_w, Wx_b, Wn_w, Wn_b, Wr_w, Wr_b, gamma, beta):
    B, N, P, D = x1.shape
    DOUT = Wx_w.shape[0]
    C = 3 * DOUT
    PB = P * N
    M = B * PB

    x1t = x1.transpose(0, 2, 1, 3).reshape(M, D)
    x2t = x2.transpose(0, 2, 1, 3).reshape(M, D)
    XA = jnp.stack([x1t, x2t])

    vspec = pl.BlockSpec(memory_space=pltpu.MemorySpace.VMEM)
    self_spec = pl.BlockSpec((1, M, D), lambda i, j: (i, 0, 0))
    other_spec = pl.BlockSpec((1, M, D), lambda i, j: (1 - i, 0, 0))
    out_spec = pl.BlockSpec(
        (1, PB, C), lambda i, j: (i * B + jnp.maximum(j - 1, 0), 0, 0))

    out = pl.pallas_call(
        _make_body(B, N, P, D, DOUT),
        grid=(2, B + 1),
        in_specs=[self_spec, other_spec] + [vspec] * 8,
        out_specs=out_spec,
        out_shape=jax.ShapeDtypeStruct((2 * B, PB, C), jnp.float32),
        scratch_shapes=[
            pltpu.VMEM((M, C), jnp.float32),
            pltpu.VMEM((1, C), jnp.float32),
            pltpu.VMEM((1, C), jnp.float32),
        ],
    )(XA, XA,
      Wx_w.T, Wn_w.T, Wr_w.T,
      Wx_b.reshape(1, DOUT), Wn_b.reshape(1, DOUT), Wr_b.reshape(1, DOUT),
      gamma.reshape(1, C), beta.reshape(1, C))

    o1 = out[:B].reshape(B, P, N, C).transpose(0, 2, 1, 3)
    o2 = out[B:].reshape(B, P, N, C).transpose(0, 2, 1, 3)
    return (o1, o2)


# speedup vs baseline: 2.6775x; 1.9102x over previous
import jax
import jax.numpy as jnp
from jax.experimental import pallas as pl

_F32 = jnp.float32


def _make_body(B, N, P, D, DOUT):
    BLK = N
    PB = P * N
    M = B * PB

    def body(x1_ref, x2_ref, wx_ref, wn_ref, wr_ref,
             bx_ref, bn_ref, br_ref, g_ref, bt_ref,
             o1_ref, o2_ref):
        att1 = []
        att2t = []
        for b in range(B):
            num = jnp.zeros((N, N), _F32)
            sq1 = jnp.zeros((N, 1), _F32)
            sq2 = jnp.zeros((N, 1), _F32)
            for p in range(P):
                r = b * PB + p * BLK
                a1 = x1_ref[r:r + BLK, :]
                a2 = x2_ref[r:r + BLK, :]
                num += jax.lax.dot_general(
                    a1, a2, (((1,), (1,)), ((), ())),
                    preferred_element_type=_F32)
                sq1 += jnp.sum(a1 * a1, axis=1, keepdims=True)
                sq2 += jnp.sum(a2 * a2, axis=1, keepdims=True)
            n1 = jnp.maximum(jnp.sqrt(sq1), 1e-6)
            n2 = jnp.maximum(jnp.sqrt(sq2), 1e-6)
            sim = num / (n1 * n2.T)
            m1 = jnp.max(sim, axis=1, keepdims=True)
            e1 = jnp.exp(sim - m1)
            att1.append(e1 / jnp.sum(e1, axis=1, keepdims=True))
            m2 = jnp.max(sim, axis=0, keepdims=True)
            e2 = jnp.exp(sim - m2)
            att2t.append(e2 / jnp.sum(e2, axis=0, keepdims=True))

        wx = wx_ref[:]
        wn = wn_ref[:]
        wr = wr_ref[:]
        bx = bx_ref[:]
        bn = bn_ref[:]
        br = br_ref[:]
        g = g_ref[:]
        bt = bt_ref[:]
        inv = 1.0 / (N - 1)

        for X, att, tr, oref in ((x1_ref[:], att1, False, o1_ref),
                                 (x2_ref[:], att2t, True, o2_ref)):
            S = jnp.dot(X, wx, preferred_element_type=_F32) + bx
            rn2 = jnp.sum(S * S, axis=1, keepdims=True)
            oref[:, 0:DOUT] = S

            Z = jnp.dot(X, wn, preferred_element_type=_F32)
            Z3 = Z.reshape(B * P, BLK, DOUT)
            s = jnp.sum(Z3, axis=1, keepdims=True)
            XN = ((s - Z3) * inv).reshape(M, DOUT) + bn
            rn2 += jnp.sum(XN * XN, axis=1, keepdims=True)
            oref[:, DOUT:2 * DOUT] = XN

            Zs = jnp.dot(X, wr, preferred_element_type=_F32)
            if not tr:
                Zo = jnp.dot(x2_ref[:], wr, preferred_element_type=_F32)
            else:
                Zo = jnp.dot(x1_ref[:], wr, preferred_element_type=_F32)
            mu_parts = []
            for b in range(B):
                a = att[b]
                for p in range(P):
                    r = b * PB + p * BLK
                    if not tr:
                        c = jnp.dot(a, Zo[r:r + BLK, :],
                                    preferred_element_type=_F32)
                    else:
                        c = jax.lax.dot_general(
                            a, Zo[r:r + BLK, :], (((0,), (0,)), ((), ())),
                            preferred_element_type=_F32)
                    mu_parts.append(Zs[r:r + BLK, :] - c + br)
            MUS = jnp.concatenate(mu_parts, axis=0)
            rn2 += jnp.sum(MUS * MUS, axis=1, keepdims=True)
            oref[:, 2 * DOUT:3 * DOUT] = MUS

            rinv = 1.0 / jnp.maximum(jnp.sqrt(rn2), 1e-12)
            h = jnp.maximum(oref[:] * rinv, 0.0)
            oref[:] = h
            sm = jnp.sum(h, axis=0, keepdims=True)
            sq = jnp.sum(h * h, axis=0, keepdims=True)
            mean = sm * (1.0 / M)
            var = jnp.maximum(sq * (1.0 / M) - mean * mean, 0.0)
            scale = g * jax.lax.rsqrt(var + 1e-5)
            shift = bt - mean * scale
            oref[:] = oref[:] * scale + shift

    return body


def kernel(x1, x2, adj1, adj2, Wx_w, Wx_b, Wn_w, Wn_b, Wr_w, Wr_b, gamma, beta):
    B, N, P, D = x1.shape
    DOUT = Wx_w.shape[0]
    C = 3 * DOUT
    M = B * N * P

    x1p = x1.transpose(0, 2, 1, 3).reshape(M, D)
    x2p = x2.transpose(0, 2, 1, 3).reshape(M, D)

    out1, out2 = pl.pallas_call(
        _make_body(B, N, P, D, DOUT),
        out_shape=(
            jax.ShapeDtypeStruct((M, C), jnp.float32),
            jax.ShapeDtypeStruct((M, C), jnp.float32),
        ),
    )(x1p, x2p,
      Wx_w.T, Wn_w.T, Wr_w.T,
      Wx_b.reshape(1, DOUT), Wn_b.reshape(1, DOUT), Wr_b.reshape(1, DOUT),
      gamma.reshape(1, C), beta.reshape(1, C))

    o1 = out1.reshape(B, P, N, C).transpose(0, 2, 1, 3)
    o2 = out2.reshape(B, P, N, C).transpose(0, 2, 1, 3)
    return (o1, o2)
